# speedup vs baseline: 1.6969x; 1.6969x over previous
import functools

import jax
import jax.numpy as jnp
from jax import lax
from jax.experimental import pallas as pl
from jax.experimental.pallas import tpu as pltpu
from jax.experimental.pallas import tpu_sc as plsc

NC = 2
NS = 16
NW = NC * NS

HID = 128
TOTAL = 1024 * 200
NROWS = TOTAL // NW
CHUNK = 128
NCHUNK = NROWS // CHUNK


def _emb_kernel(ids_hbm, pids_hbm, word_hbm, pos_hbm, out_hbm,
                idx_v, pidx_v, wbuf, pbuf, wsem, psem):
    wid = lax.axis_index("s") * NC + lax.axis_index("c")

    pltpu.sync_copy(ids_hbm.at[wid], idx_v)
    pltpu.sync_copy(pids_hbm.at[wid], pidx_v)

    @pl.loop(0, NCHUNK)
    def chunk_loop(i):
        pltpu.async_copy(word_hbm.at[idx_v.at[i]], wbuf, wsem).wait()
        pltpu.async_copy(pos_hbm.at[pidx_v.at[i]], pbuf, psem).wait()

        @pl.loop(0, CHUNK)
        def row_loop(r):
            for c in range(HID // 16):
                x = pbuf[r, pl.ds(c * 16, 16)]
                plsc.addupdate(wbuf.at[r, pl.ds(c * 16, 16)], x)

        pltpu.sync_copy(wbuf, out_hbm.at[wid, i])


@jax.jit
def _run(ids3, pids3, word_embeddings, pos_embeddings):
    mesh = plsc.VectorSubcoreMesh(core_axis_name="c", subcore_axis_name="s")
    k = functools.partial(
        pl.kernel,
        out_type=jax.ShapeDtypeStruct((NW, NCHUNK, CHUNK, HID), jnp.float32),
        mesh=mesh,
        scratch_types=[
            pltpu.VMEM((NCHUNK, CHUNK), jnp.int32),
            pltpu.VMEM((NCHUNK, CHUNK), jnp.int32),
            pltpu.VMEM((CHUNK, HID), jnp.float32),
            pltpu.VMEM((CHUNK, HID), jnp.float32),
            pltpu.SemaphoreType.DMA,
            pltpu.SemaphoreType.DMA,
        ],
    )(_emb_kernel)
    return k(ids3, pids3, word_embeddings, pos_embeddings)


def kernel(input_ids, pos_ids, word_embeddings, position_embeddings,
           token_type_embeddings, pos_embeddings):
    del position_embeddings, token_type_embeddings
    B, S = input_ids.shape
    ids3 = input_ids.reshape(NW, NCHUNK, CHUNK).astype(jnp.int32)
    pids3 = pos_ids.reshape(NW, NCHUNK, CHUNK).astype(jnp.int32)
    out = _run(ids3, pids3, word_embeddings, pos_embeddings)
    return out.reshape(B, S, HID)
